# baseline (speedup 1.0000x reference)
import functools

import jax
import jax.numpy as jnp
from jax import lax
from jax.experimental import pallas as pl
from jax.experimental.pallas import tpu as pltpu
from jax.experimental.pallas import tpu_sc as plsc

B_TOK = 16384
S_TOK = 50
D = 64
B = B_TOK * S_TOK

_info = plsc.get_sparse_core_info()
NC = _info.num_cores
NS = _info.num_subcores
NW = NC * NS
LB = 128
GROUP = 512
BLOCKS_PER_GROUP = GROUP // LB
N_BLOCKS = B // LB
N_GROUPS = B // GROUP
G_PER_W = N_GROUPS // NW


def _k0_flatten(ids_t_hbm, out_hbm, vbuf):
    wid = lax.axis_index("s") * NC + lax.axis_index("c")
    for rep in range(2):
        s = wid + NW * rep

        @pl.when(s < S_TOK)
        def _():
            pltpu.sync_copy(ids_t_hbm.at[s], vbuf)
            pltpu.sync_copy(vbuf, out_hbm.at[pl.ds(s * B_TOK, B_TOK)])


def _k2_gather(table_hbm, ids2_hbm, out_hbm, idx_all, rows_v, rowsT,
               gsem, ssem):
    wid = lax.axis_index("s") * NC + lax.axis_index("c")
    g0 = G_PER_W * wid

    iota16 = jnp.arange(16, dtype=jnp.int32)
    lidx = [iota16 + 16 * c for c in range(8)]

    pltpu.sync_copy(ids2_hbm.at[pl.ds(g0, G_PER_W)], idx_all)

    def start_gather(g, b):
        pltpu.async_copy(table_hbm.at[idx_all.at[g]], rows_v.at[b],
                         gsem.at[b])

    def gather_descr(b):
        return pltpu.make_async_copy(table_hbm.at[idx_all.at[0]],
                                     rows_v.at[b], gsem.at[b])

    def store_descr(tb):
        return pltpu.make_async_copy(
            rowsT.at[tb],
            out_hbm.at[pl.ds(0, 1), slice(None), pl.ds(0, 1)],
            ssem.at[tb])

    def transpose_block(b, blk, tb):
        rows_blk = rows_v.at[b, pl.ds(LB * blk, LB)]

        @plsc.parallel_loop(0, D, unroll=2)
        def _(d):
            dsplat = jnp.full((16,), 0, dtype=jnp.int32) + d
            fg = d // 8
            off = (d % 8) * 128
            vs = [plsc.load_gather(rows_blk, [lidx[c], dsplat])
                  for c in range(8)]
            for c in range(8):
                rowsT[tb, 0, fg, 0, pl.ds(off + 16 * c, 16)] = vs[c]

    def store_block(g, blk, tb):
        gid = BLOCKS_PER_GROUP * (g0 + g) + blk
        s = gid // 128
        bt = gid % 128
        pltpu.async_copy(
            rowsT.at[tb],
            out_hbm.at[pl.ds(s, 1), slice(None), pl.ds(bt, 1)],
            ssem.at[tb])

    start_gather(0, 0)

    def outer(h, carry):
        for sub in range(2):
            g = 2 * h + sub
            b = sub
            gather_descr(b).wait()

            @pl.when(g + 1 < G_PER_W)
            def _():
                start_gather(g + 1, 1 - b)

            for blk in range(BLOCKS_PER_GROUP):
                tb = blk % 2
                if blk >= 2:
                    store_descr(tb).wait()
                else:
                    @pl.when(g > 0)
                    def _():
                        store_descr(tb).wait()
                transpose_block(b, blk, tb)
                store_block(g, blk, tb)
        return carry

    lax.fori_loop(0, G_PER_W // 2, outer, 0)

    for tb in range(2):
        store_descr(tb).wait()


def kernel(token_ids, weight):
    mesh = plsc.VectorSubcoreMesh(core_axis_name="c", subcore_axis_name="s")

    ids_t = token_ids.T

    k0 = functools.partial(
        pl.kernel,
        mesh=mesh,
        out_type=jax.ShapeDtypeStruct((B,), jnp.int32),
        scratch_types=[pltpu.VMEM((B_TOK,), jnp.int32)],
    )(_k0_flatten)
    ids_flat = k0(ids_t.astype(jnp.int32))
    ids2 = ids_flat.reshape(N_GROUPS, GROUP)

    k2 = functools.partial(
        pl.kernel,
        mesh=mesh,
        out_type=jax.ShapeDtypeStruct((S_TOK, 8, 128, 1024), jnp.float32),
        scratch_types=[
            pltpu.VMEM((G_PER_W, GROUP), jnp.int32),
            pltpu.VMEM((2, GROUP, D), jnp.float32),
            pltpu.VMEM((2, 1, 8, 1, 1024), jnp.float32),
            pltpu.SemaphoreType.DMA((2,)),
            pltpu.SemaphoreType.DMA((2,)),
        ],
        compiler_params=pltpu.CompilerParams(use_tc_tiling_on_sc=False,
                                             needs_layout_passes=False),
    )(_k2_gather)
    out4 = k2(weight, ids2)

    out5 = out4.reshape(S_TOK, 8, 128, 8, 128)
    return jnp.transpose(out5, (2, 4, 0, 1, 3)).reshape(B_TOK, S_TOK, D)


# speedup vs baseline: 1.9743x; 1.9743x over previous
import functools

import jax
import jax.numpy as jnp
from jax import lax
from jax.experimental import pallas as pl
from jax.experimental.pallas import tpu as pltpu
from jax.experimental.pallas import tpu_sc as plsc

B_TOK = 16384
S_TOK = 50
D = 64
B = B_TOK * S_TOK

_info = plsc.get_sparse_core_info()
NC = _info.num_cores
NS = _info.num_subcores
NW = NC * NS
LB = 128
GROUP = 512
BLOCKS_PER_GROUP = GROUP // LB
N_BLOCKS = B // LB
N_GROUPS = B // GROUP
G_PER_W = N_GROUPS // NW


def _k0_flatten(ids_t_hbm, out_hbm, vbuf):
    wid = lax.axis_index("s") * NC + lax.axis_index("c")
    for rep in range(2):
        s = wid + NW * rep

        @pl.when(s < S_TOK)
        def _():
            pltpu.sync_copy(ids_t_hbm.at[s], vbuf)
            pltpu.sync_copy(vbuf, out_hbm.at[pl.ds(s * B_TOK, B_TOK)])


def _k2_gather(table_hbm, ids2_hbm, out_hbm, idx_all, rows_v, rows_pad,
               rowsT, gsem, ssem):
    wid = lax.axis_index("s") * NC + lax.axis_index("c")
    g0 = G_PER_W * wid

    iota16 = jnp.arange(16, dtype=jnp.int32)
    lidx = [iota16 + 16 * c for c in range(8)]

    pltpu.sync_copy(ids2_hbm.at[pl.ds(g0, G_PER_W)], idx_all)

    def start_gather(g, b):
        pltpu.async_copy(table_hbm.at[idx_all.at[g]], rows_v.at[b],
                         gsem.at[b])

    def gather_descr(b):
        return pltpu.make_async_copy(table_hbm.at[idx_all.at[0]],
                                     rows_v.at[b], gsem.at[b])

    def store_descr(tb):
        return pltpu.make_async_copy(
            rowsT.at[tb],
            out_hbm.at[pl.ds(0, 1), slice(None), pl.ds(0, 1)],
            ssem.at[tb])

    def transpose_block(b, blk, tb):
        @plsc.parallel_loop(0, LB, unroll=4)
        def _(l):
            for c in range(4):
                rows_pad[l, pl.ds(16 * c, 16)] = (
                    rows_v[b, LB * blk + l, pl.ds(16 * c, 16)])

        @plsc.parallel_loop(0, D, unroll=4)
        def _(d):
            dsplat = jnp.full((16,), 0, dtype=jnp.int32) + d
            fg = d // 8
            off = (d % 8) * 128
            for c in range(8):
                v = plsc.load_gather(rows_pad, [lidx[c], dsplat])
                rowsT[tb, 0, fg, 0, pl.ds(off + 16 * c, 16)] = v

    def store_block(g, blk, tb):
        gid = BLOCKS_PER_GROUP * (g0 + g) + blk
        s = gid // 128
        bt = gid % 128
        pltpu.async_copy(
            rowsT.at[tb],
            out_hbm.at[pl.ds(s, 1), slice(None), pl.ds(bt, 1)],
            ssem.at[tb])

    start_gather(0, 0)

    def outer(h, carry):
        for sub in range(2):
            g = 2 * h + sub
            b = sub
            gather_descr(b).wait()

            @pl.when(g + 1 < G_PER_W)
            def _():
                start_gather(g + 1, 1 - b)

            for blk in range(BLOCKS_PER_GROUP):
                tb = blk % 2
                if blk >= 2:
                    store_descr(tb).wait()
                else:
                    @pl.when(g > 0)
                    def _():
                        store_descr(tb).wait()
                transpose_block(b, blk, tb)
                store_block(g, blk, tb)
        return carry

    lax.fori_loop(0, G_PER_W // 2, outer, 0)

    for tb in range(2):
        store_descr(tb).wait()


def kernel(token_ids, weight):
    mesh = plsc.VectorSubcoreMesh(core_axis_name="c", subcore_axis_name="s")

    ids_t = token_ids.T

    k0 = functools.partial(
        pl.kernel,
        mesh=mesh,
        out_type=jax.ShapeDtypeStruct((B,), jnp.int32),
        scratch_types=[pltpu.VMEM((B_TOK,), jnp.int32)],
    )(_k0_flatten)
    ids_flat = k0(ids_t.astype(jnp.int32))
    ids2 = ids_flat.reshape(N_GROUPS, GROUP)

    k2 = functools.partial(
        pl.kernel,
        mesh=mesh,
        out_type=jax.ShapeDtypeStruct((S_TOK, 8, 128, 1024), jnp.float32),
        scratch_types=[
            pltpu.VMEM((G_PER_W, GROUP), jnp.int32),
            pltpu.VMEM((2, GROUP, D), jnp.float32),
            pltpu.VMEM((LB, 65), jnp.float32),
            pltpu.VMEM((2, 1, 8, 1, 1024), jnp.float32),
            pltpu.SemaphoreType.DMA((2,)),
            pltpu.SemaphoreType.DMA((2,)),
        ],
        compiler_params=pltpu.CompilerParams(use_tc_tiling_on_sc=False,
                                             needs_layout_passes=False),
    )(_k2_gather)
    out4 = k2(weight, ids2)

    out5 = out4.reshape(S_TOK, 8, 128, 8, 128)
    return jnp.transpose(out5, (2, 4, 0, 1, 3)).reshape(B_TOK, S_TOK, D)
